# Initial kernel scaffold; baseline (speedup 1.0000x reference)
import jax
import jax.numpy as jnp
from jax.experimental import pallas as pl


def kernel(x, prototypes):
    raise NotImplementedError("write your pallas kernel here")


# speedup vs baseline: 1.4036x; 1.4036x over previous
import functools

import jax
import jax.numpy as jnp
from jax.experimental import pallas as pl
from jax.experimental.pallas import tpu as pltpu

B = 8
S = 1024
D = 768
P = 4096

P_BLK = 2048
S_CHUNK = 256
N_PT = P // P_BLK
N_CHUNK = S // S_CHUNK

_PREC = jax.lax.Precision.DEFAULT


def _proto_kernel(x_ref, p_ref, dist_ref, idx_ref, pn_ref):
    b = pl.program_id(1)

    @pl.when(b == 0)
    def _():
        pblk = p_ref[...]
        ss = jnp.sum(pblk * pblk, axis=1, keepdims=True)
        pn_ref[...] = pblk * jax.lax.rsqrt(jnp.maximum(ss, 1e-24))

    xblk = x_ref[0]
    ssx = jnp.sum(xblk * xblk, axis=1, keepdims=True)
    xn = xblk * jax.lax.rsqrt(jnp.maximum(ssx, 1e-24))

    pn = pn_ref[...]

    m = None
    mi = None
    for c in range(N_CHUNK):
        xc = xn[c * S_CHUNK:(c + 1) * S_CHUNK]
        dots = jax.lax.dot_general(
            xc, pn,
            dimension_numbers=(((1,), (1,)), ((), ())),
            precision=_PREC,
            preferred_element_type=jnp.float32,
        )
        dist = 1.0 - dots
        cmin = jnp.min(dist, axis=0, keepdims=True)
        rows = jax.lax.broadcasted_iota(jnp.int32, dist.shape, 0) + c * S_CHUNK
        cidx = jnp.min(
            jnp.where(dist == cmin, rows, jnp.int32(S)),
            axis=0, keepdims=True)
        if c == 0:
            m, mi = cmin, cidx
        else:
            upd = cmin < m
            mi = jnp.where(upd, cidx, mi)
            m = jnp.minimum(cmin, m)

    dist_ref[0] = m
    idx_ref[0] = mi


@jax.jit
def kernel(x, prototypes):
    grid = (N_PT, B)
    dist, idx = pl.pallas_call(
        _proto_kernel,
        grid=grid,
        in_specs=[
            pl.BlockSpec((1, S, D), lambda pt, b: (b, 0, 0)),
            pl.BlockSpec((P_BLK, D), lambda pt, b: (pt, 0)),
        ],
        out_specs=[
            pl.BlockSpec((1, 1, P_BLK), lambda pt, b: (b, 0, pt)),
            pl.BlockSpec((1, 1, P_BLK), lambda pt, b: (b, 0, pt)),
        ],
        out_shape=[
            jax.ShapeDtypeStruct((B, 1, P), jnp.float32),
            jax.ShapeDtypeStruct((B, 1, P), jnp.int32),
        ],
        scratch_shapes=[pltpu.VMEM((P_BLK, D), jnp.float32)],
    )(x, prototypes)
    return dist, idx.astype(jnp.int64)


# speedup vs baseline: 1.6308x; 1.1619x over previous
import functools

import jax
import jax.numpy as jnp
from jax.experimental import pallas as pl
from jax.experimental.pallas import tpu as pltpu

B = 8
S = 1024
D = 768
P = 4096

P_BLK = 2048
S_CHUNK = 256
N_PT = P // P_BLK
N_CHUNK = S // S_CHUNK

_PREC = jax.lax.Precision.DEFAULT


def _proto_kernel(x_ref, p_ref, dist_ref, idx_ref, pn_ref):
    b = pl.program_id(1)

    @pl.when(b == 0)
    def _():
        pblk = p_ref[...]
        ss = jnp.sum(pblk * pblk, axis=1, keepdims=True)
        pn_ref[...] = pblk * jax.lax.rsqrt(jnp.maximum(ss, 1e-24))

    xblk = x_ref[0]
    ssx = jnp.sum(xblk * xblk, axis=1, keepdims=True)
    xn = xblk * jax.lax.rsqrt(jnp.maximum(ssx, 1e-24))

    pn = pn_ref[...]

    m = None
    mi = None
    for c in range(N_CHUNK):
        xc = xn[c * S_CHUNK:(c + 1) * S_CHUNK]
        dots = jax.lax.dot_general(
            xc, pn,
            dimension_numbers=(((1,), (1,)), ((), ())),
            precision=_PREC,
            preferred_element_type=jnp.float32,
        )
        cmax = jnp.max(dots, axis=0, keepdims=True)
        cidx = (jnp.argmax(dots, axis=0, keepdims=True).astype(jnp.int32)
                + c * S_CHUNK)
        if c == 0:
            m, mi = cmax, cidx
        else:
            upd = cmax > m
            mi = jnp.where(upd, cidx, mi)
            m = jnp.maximum(cmax, m)

    dist_ref[0] = 1.0 - m
    idx_ref[0] = mi


@jax.jit
def kernel(x, prototypes):
    grid = (N_PT, B)
    dist, idx = pl.pallas_call(
        _proto_kernel,
        grid=grid,
        in_specs=[
            pl.BlockSpec((1, S, D), lambda pt, b: (b, 0, 0)),
            pl.BlockSpec((P_BLK, D), lambda pt, b: (pt, 0)),
        ],
        out_specs=[
            pl.BlockSpec((1, 1, P_BLK), lambda pt, b: (b, 0, pt)),
            pl.BlockSpec((1, 1, P_BLK), lambda pt, b: (b, 0, pt)),
        ],
        out_shape=[
            jax.ShapeDtypeStruct((B, 1, P), jnp.float32),
            jax.ShapeDtypeStruct((B, 1, P), jnp.int32),
        ],
        scratch_shapes=[pltpu.VMEM((P_BLK, D), jnp.float32)],
    )(x, prototypes)
    return dist, idx.astype(jnp.int64)
